# baseline (speedup 1.0000x reference)
import itertools

import jax
import jax.numpy as jnp
import numpy as np
from jax import lax
from jax.experimental import pallas as pl
from jax.experimental.pallas import tpu as pltpu
from jax.experimental.pallas import tpu_sc as plsc

NUM_BITS = 32
BATCH = 16384
NTERMS = 5488

_N1 = 32
_PAIRS = list(itertools.combinations(range(NUM_BITS), 2))
_N2 = len(_PAIRS)
_C3 = np.array(list(itertools.combinations(range(NUM_BITS), 3)), np.int32)
_N3 = _C3.shape[0]

SC_ROWS = 512
TC_ROWS = BATCH - SC_ROWS
NW = 32
ROWS_PER_W = SC_ROWS // NW
BLK = 16
NBLK = ROWS_PER_W // BLK


_NCOL = 640
_PAIR_COL = {(int(j), int(k)): p for p, (j, k) in enumerate(_PAIRS)}

_G3 = np.zeros((NUM_BITS, _NCOL), np.int32)
for t, (i, j, k) in enumerate(_C3):
    _G3[i, _PAIR_COL[(int(j), int(k))]] = 16 + _N1 + _N2 + t
for i in range(_N1):
    _G3[i, _N2 + i] = 16 + i

_trip_idx = {t: n for n, t in enumerate(itertools.combinations(range(NUM_BITS), 3))}
_G_SC = np.concatenate([
    16 + np.arange(_N1 + _N2, dtype=np.int32),
    np.array([16 + _N1 + _N2 + _trip_idx[(i, j, k)]
              for (j, k) in _PAIRS for i in range(j)], np.int32),
])

_W3MAT_LEN = NUM_BITS * _NCOL
_SC_W_OFF = _W3MAT_LEN
_G_ALL = np.concatenate([_G3.ravel(), _G_SC])
_STAGE_LEN = -(-_G_ALL.size // (NW * 16)) * (NW * 16)
_G_ALL = np.concatenate([_G_ALL, np.zeros(_STAGE_LEN - _G_ALL.size, np.int32)])
_CHUNK = _STAGE_LEN // NW
_NGATH = _CHUNK // 16


def _stage_body(w_hbm, g_hbm, out_hbm, w_v, g_v, o_v):
    wid = lax.axis_index("s") * 2 + lax.axis_index("c")
    base = wid * _CHUNK
    w_v[pl.ds(0, 16)] = jnp.zeros((16,), jnp.float32)
    pltpu.sync_copy(w_hbm, w_v.at[pl.ds(16, NTERMS)])
    pltpu.sync_copy(g_hbm.at[pl.ds(base, _CHUNK)], g_v)

    def body(t, _):
        idx = g_v[pl.ds(t * 16, 16)]
        o_v[pl.ds(t * 16, 16)] = plsc.load_gather(w_v, [idx])
        return 0

    lax.fori_loop(0, _NGATH, body, 0)
    pltpu.sync_copy(o_v, out_hbm.at[pl.ds(base, _CHUNK)])


def _stage_weights(w):
    run = pl.kernel(
        _stage_body,
        out_type=jax.ShapeDtypeStruct((_STAGE_LEN,), jnp.float32),
        mesh=plsc.VectorSubcoreMesh(core_axis_name="c", subcore_axis_name="s"),
        compiler_params=pltpu.CompilerParams(needs_layout_passes=False),
        scratch_types=[
            pltpu.VMEM((16 + NTERMS,), jnp.float32),
            pltpu.VMEM((_CHUNK,), jnp.int32),
            pltpu.VMEM((_CHUNK,), jnp.float32),
        ],
    )
    return run(w, jnp.asarray(_G_ALL))


_EJEK = np.zeros((NUM_BITS, 2 * _NCOL), np.float32)
for p, (j, k) in enumerate(_PAIRS):
    _EJEK[j, p] = 1.0
    _EJEK[k, _NCOL + p] = 1.0
for i in range(_N1):
    _EJEK[i, _N2 + i] = 1.0
    _EJEK[i, _NCOL + _N2 + i] = 1.0

_BBLK = 1984


def _tc_body(x_ref, ejek_ref, w3_ref, w2_ref, out_ref):
    s = (1 - 2 * x_ref[...]).astype(jnp.bfloat16)
    prod = jax.lax.dot_general(
        s, ejek_ref[...], (((1,), (0,)), ((), ())),
        preferred_element_type=jnp.float32,
    )
    a = jax.lax.dot_general(
        s, w3_ref[...].astype(jnp.bfloat16), (((1,), (0,)), ((), ())),
        preferred_element_type=jnp.float32,
    )
    sj = prod[:, :_NCOL]
    sk = prod[:, _NCOL:]
    out_ref[...] = jnp.sum(sj * sk * (a + w2_ref[...]), axis=1, keepdims=True)


def _tc_half(x, w3mat, w2pad):
    grid = TC_ROWS // _BBLK
    out = pl.pallas_call(
        _tc_body,
        grid=(grid,),
        in_specs=[
            pl.BlockSpec((_BBLK, NUM_BITS), lambda i: (i, 0)),
            pl.BlockSpec((NUM_BITS, 2 * _NCOL), lambda i: (0, 0)),
            pl.BlockSpec((NUM_BITS, _NCOL), lambda i: (0, 0)),
            pl.BlockSpec((1, _NCOL), lambda i: (0, 0)),
        ],
        out_specs=pl.BlockSpec((_BBLK, 1), lambda i: (i, 0)),
        out_shape=jax.ShapeDtypeStruct((TC_ROWS, 1), jnp.float32),
        compiler_params=pltpu.CompilerParams(
            dimension_semantics=("arbitrary",),
        ),
    )(x, jnp.asarray(_EJEK, dtype=jnp.bfloat16), w3mat, w2pad)
    return out[:, 0]


def _sc_body(x_hbm, staged_hbm, out_hbm, x_v, w_v, st_v, out_v):
    wid = lax.axis_index("s") * 2 + lax.axis_index("c")
    base = wid * ROWS_PER_W
    pltpu.sync_copy(x_hbm.at[pl.ds(base, ROWS_PER_W)], x_v)
    pltpu.sync_copy(staged_hbm.at[pl.ds(_SC_W_OFF, NTERMS)], w_v)
    lanes = lax.iota(jnp.int32, 16)

    def splat_w(idx):
        return plsc.load_gather(w_v, [jnp.full((16,), idx, jnp.int32)])

    def block_body(b, _):
        rb = b * BLK
        rows = rb + lanes

        def load_i(i, acc):
            g = plsc.load_gather(x_v, [rows, jnp.full((16,), i, jnp.int32)])
            s = (1 - 2 * g).astype(jnp.float32)
            st_v[pl.ds(i * BLK, BLK)] = s
            return acc + splat_w(i) * s

        acc = lax.fori_loop(0, NUM_BITS, load_i, jnp.zeros((16,), jnp.float32))

        p2_base = _N1
        p3_base = _N1 + _N2
        for j in range(NUM_BITS):
            sj = st_v[pl.ds(j * BLK, BLK)]

            def k_body(k, acck, j=j, sj=sj, p2b=p2_base, p3b=p3_base):
                sk = st_v[pl.ds(k * BLK, BLK)]
                d = k - j - 1

                def i_body(i, inner):
                    return inner + splat_w(p3b + d * j + i) * st_v[pl.ds(i * BLK, BLK)]

                inner = lax.fori_loop(0, j, i_body, splat_w(p2b + d), unroll=4)
                return acck + sj * sk * inner

            acc = lax.fori_loop(j + 1, NUM_BITS, k_body, acc, unroll=2)
            p2_base += NUM_BITS - 1 - j
            p3_base += (NUM_BITS - 1 - j) * j
        out_v[pl.ds(rb, BLK)] = acc
        return 0

    lax.fori_loop(0, NBLK, block_body, 0)
    pltpu.sync_copy(out_v, out_hbm.at[pl.ds(wid * ROWS_PER_W, ROWS_PER_W)])


def _sc_half(x, staged):
    run = pl.kernel(
        _sc_body,
        out_type=jax.ShapeDtypeStruct((SC_ROWS,), jnp.float32),
        mesh=plsc.VectorSubcoreMesh(core_axis_name="c", subcore_axis_name="s"),
        compiler_params=pltpu.CompilerParams(needs_layout_passes=False),
        scratch_types=[
            pltpu.VMEM((ROWS_PER_W, NUM_BITS), jnp.int32),
            pltpu.VMEM((NTERMS,), jnp.float32),
            pltpu.VMEM((NUM_BITS * BLK,), jnp.float32),
            pltpu.VMEM((ROWS_PER_W,), jnp.float32),
        ],
    )
    return run(x, staged)


@jax.jit
def kernel(inputs, kernel):
    staged = _stage_weights(kernel)
    e_sc = _sc_half(inputs[TC_ROWS:], staged)
    w3mat = staged[:_W3MAT_LEN].reshape(NUM_BITS, _NCOL)
    w2pad = jnp.pad(kernel[_N1:_N1 + _N2], (0, _NCOL - _N2)).reshape(1, _NCOL)
    e_tc = _tc_half(inputs, w3mat, w2pad)
    return jnp.concatenate([e_tc, e_sc])


# speedup vs baseline: 1.0035x; 1.0035x over previous
import itertools

import jax
import jax.numpy as jnp
import numpy as np
from jax import lax
from jax.experimental import pallas as pl
from jax.experimental.pallas import tpu as pltpu
from jax.experimental.pallas import tpu_sc as plsc

NUM_BITS = 32
BATCH = 16384
NTERMS = 5488

_N1 = 32
_PAIRS = list(itertools.combinations(range(NUM_BITS), 2))
_N2 = len(_PAIRS)
_C3 = np.array(list(itertools.combinations(range(NUM_BITS), 3)), np.int32)
_N3 = _C3.shape[0]

SC_ROWS = 512
TC_ROWS = BATCH - SC_ROWS
NW = 32
ROWS_PER_W = SC_ROWS // NW
BLK = 16
NBLK = ROWS_PER_W // BLK


_NCOL = 640
_PAIR_COL = {(int(j), int(k)): p for p, (j, k) in enumerate(_PAIRS)}

_G3 = np.zeros((NUM_BITS, _NCOL), np.int32)
for t, (i, j, k) in enumerate(_C3):
    _G3[i, _PAIR_COL[(int(j), int(k))]] = 16 + _N1 + _N2 + t
for i in range(_N1):
    _G3[i, _N2 + i] = 16 + i

_trip_idx = {t: n for n, t in enumerate(itertools.combinations(range(NUM_BITS), 3))}
_G_SC = np.concatenate([
    16 + np.arange(_N1 + _N2, dtype=np.int32),
    np.array([16 + _N1 + _N2 + _trip_idx[(i, j, k)]
              for (j, k) in _PAIRS for i in range(j)], np.int32),
])

_W3MAT_LEN = NUM_BITS * _NCOL
_SC_W_OFF = _W3MAT_LEN
_G_ALL = np.concatenate([_G3.ravel(), _G_SC])
_STAGE_LEN = -(-_G_ALL.size // (NW * 16)) * (NW * 16)
_G_ALL = np.concatenate([_G_ALL, np.zeros(_STAGE_LEN - _G_ALL.size, np.int32)])
_CHUNK = _STAGE_LEN // NW
_NGATH = _CHUNK // 16


def _stage_body(w_hbm, g_hbm, out_hbm, w_v, g_v, o_v):
    wid = lax.axis_index("s") * 2 + lax.axis_index("c")
    base = wid * _CHUNK
    w_v[pl.ds(0, 16)] = jnp.zeros((16,), jnp.float32)
    pltpu.sync_copy(w_hbm, w_v.at[pl.ds(16, NTERMS)])
    pltpu.sync_copy(g_hbm.at[pl.ds(base, _CHUNK)], g_v)

    def body(t, _):
        idx = g_v[pl.ds(t * 16, 16)]
        o_v[pl.ds(t * 16, 16)] = plsc.load_gather(w_v, [idx])
        return 0

    lax.fori_loop(0, _NGATH, body, 0)
    pltpu.sync_copy(o_v, out_hbm.at[pl.ds(base, _CHUNK)])


def _stage_weights(w):
    run = pl.kernel(
        _stage_body,
        out_type=jax.ShapeDtypeStruct((_STAGE_LEN,), jnp.float32),
        mesh=plsc.VectorSubcoreMesh(core_axis_name="c", subcore_axis_name="s"),
        compiler_params=pltpu.CompilerParams(needs_layout_passes=False),
        scratch_types=[
            pltpu.VMEM((16 + NTERMS,), jnp.float32),
            pltpu.VMEM((_CHUNK,), jnp.int32),
            pltpu.VMEM((_CHUNK,), jnp.float32),
        ],
    )
    return run(w, jnp.asarray(_G_ALL))


_EJEK = np.zeros((NUM_BITS, 2 * _NCOL), np.float32)
for p, (j, k) in enumerate(_PAIRS):
    _EJEK[j, p] = 1.0
    _EJEK[k, _NCOL + p] = 1.0
for i in range(_N1):
    _EJEK[i, _N2 + i] = 1.0
    _EJEK[i, _NCOL + _N2 + i] = 1.0

_BBLK = 1984


def _tc_body(x_ref, ejek_ref, w3_ref, w2_ref, out_ref):
    s = (1 - 2 * x_ref[...]).astype(jnp.float32)
    prod = jax.lax.dot_general(
        s, ejek_ref[...], (((1,), (0,)), ((), ())),
        preferred_element_type=jnp.float32,
    )
    a = jax.lax.dot_general(
        s, w3_ref[...], (((1,), (0,)), ((), ())),
        preferred_element_type=jnp.float32,
    )
    sj = prod[:, :_NCOL]
    sk = prod[:, _NCOL:]
    out_ref[...] = jnp.sum(sj * sk * (a + w2_ref[...]), axis=1, keepdims=True)


def _tc_half(x, w3mat, w2pad):
    grid = TC_ROWS // _BBLK
    out = pl.pallas_call(
        _tc_body,
        grid=(grid,),
        in_specs=[
            pl.BlockSpec((_BBLK, NUM_BITS), lambda i: (i, 0)),
            pl.BlockSpec((NUM_BITS, 2 * _NCOL), lambda i: (0, 0)),
            pl.BlockSpec((NUM_BITS, _NCOL), lambda i: (0, 0)),
            pl.BlockSpec((1, _NCOL), lambda i: (0, 0)),
        ],
        out_specs=pl.BlockSpec((_BBLK, 1), lambda i: (i, 0)),
        out_shape=jax.ShapeDtypeStruct((TC_ROWS, 1), jnp.float32),
        compiler_params=pltpu.CompilerParams(
            dimension_semantics=("arbitrary",),
        ),
    )(x, jnp.asarray(_EJEK), w3mat, w2pad)
    return out[:, 0]


def _sc_body(x_hbm, staged_hbm, out_hbm, x_v, w_v, st_v, out_v):
    wid = lax.axis_index("s") * 2 + lax.axis_index("c")
    base = wid * ROWS_PER_W
    pltpu.sync_copy(x_hbm.at[pl.ds(base, ROWS_PER_W)], x_v)
    pltpu.sync_copy(staged_hbm.at[pl.ds(_SC_W_OFF, NTERMS)], w_v)
    lanes = lax.iota(jnp.int32, 16)

    def splat_w(idx):
        return plsc.load_gather(w_v, [jnp.full((16,), idx, jnp.int32)])

    def block_body(b, _):
        rb = b * BLK
        rows = rb + lanes

        def load_i(i, acc):
            g = plsc.load_gather(x_v, [rows, jnp.full((16,), i, jnp.int32)])
            s = (1 - 2 * g).astype(jnp.float32)
            st_v[pl.ds(i * BLK, BLK)] = s
            return acc + splat_w(i) * s

        acc = lax.fori_loop(0, NUM_BITS, load_i, jnp.zeros((16,), jnp.float32))

        p2_base = _N1
        p3_base = _N1 + _N2
        for j in range(NUM_BITS):
            sj = st_v[pl.ds(j * BLK, BLK)]

            def k_body(k, acck, j=j, sj=sj, p2b=p2_base, p3b=p3_base):
                sk = st_v[pl.ds(k * BLK, BLK)]
                d = k - j - 1

                def i_body(i, inner):
                    return inner + splat_w(p3b + d * j + i) * st_v[pl.ds(i * BLK, BLK)]

                inner = lax.fori_loop(0, j, i_body, splat_w(p2b + d), unroll=4)
                return acck + sj * sk * inner

            acc = lax.fori_loop(j + 1, NUM_BITS, k_body, acc, unroll=2)
            p2_base += NUM_BITS - 1 - j
            p3_base += (NUM_BITS - 1 - j) * j
        out_v[pl.ds(rb, BLK)] = acc
        return 0

    lax.fori_loop(0, NBLK, block_body, 0)
    pltpu.sync_copy(out_v, out_hbm.at[pl.ds(wid * ROWS_PER_W, ROWS_PER_W)])


def _sc_half(x, staged):
    run = pl.kernel(
        _sc_body,
        out_type=jax.ShapeDtypeStruct((SC_ROWS,), jnp.float32),
        mesh=plsc.VectorSubcoreMesh(core_axis_name="c", subcore_axis_name="s"),
        compiler_params=pltpu.CompilerParams(needs_layout_passes=False),
        scratch_types=[
            pltpu.VMEM((ROWS_PER_W, NUM_BITS), jnp.int32),
            pltpu.VMEM((NTERMS,), jnp.float32),
            pltpu.VMEM((NUM_BITS * BLK,), jnp.float32),
            pltpu.VMEM((ROWS_PER_W,), jnp.float32),
        ],
    )
    return run(x, staged)


@jax.jit
def kernel(inputs, kernel):
    staged = _stage_weights(kernel)
    e_sc = _sc_half(inputs[TC_ROWS:], staged)
    w3mat = staged[:_W3MAT_LEN].reshape(NUM_BITS, _NCOL)
    w2pad = jnp.pad(kernel[_N1:_N1 + _N2], (0, _NCOL - _N2)).reshape(1, _NCOL)
    e_tc = _tc_half(inputs, w3mat, w2pad)
    return jnp.concatenate([e_tc, e_sc])
